# baseline (speedup 1.0000x reference)
import functools

import jax
import jax.numpy as jnp
from jax import lax
from jax.experimental import pallas as pl
from jax.experimental.pallas import tpu as pltpu
from jax.experimental.pallas import tpu_sc as plsc

N = 10000
F = 128
FP = 144
E = 320000

NC = 2
NS = 16
NW = NC * NS
EPT = E // NW
K = 40
NB = EPT // K
NP = 10112
RPT = NP // NS
WCH = 40

_mesh = plsc.VectorSubcoreMesh(core_axis_name="c", subcore_axis_name="s")


@functools.partial(
    pl.kernel,
    out_type=jax.ShapeDtypeStruct((NC, NP, FP), jnp.float32),
    mesh=_mesh,
    compiler_params=pltpu.CompilerParams(use_tc_tiling_on_sc=False),
    scratch_types=(
        tuple(pltpu.VMEM((K,), jnp.int32) for _ in range(4)),
        tuple(pltpu.VMEM((K,), jnp.int32) for _ in range(4)),
        tuple(pltpu.VMEM((K, FP), jnp.float32) for _ in range(4)),
        pltpu.VMEM_SHARED((NP, FP), jnp.float32),
        tuple(pltpu.SemaphoreType.DMA for _ in range(4)),
        tuple(pltpu.SemaphoreType.DMA for _ in range(4)),
        tuple(pltpu.SemaphoreType.DMA for _ in range(4)),
    ),
)
def _sc_scatter(ei_hbm, xp_hbm, nsum_hbm,
                colbs, rowbs, rows, nsum_sh, semi, semg, sems):
    c = lax.axis_index("c")
    s = lax.axis_index("s")
    wid = s * NC + c
    ebase = wid * EPT
    base = s * RPT

    zero16 = jnp.zeros((16,), jnp.float32)
    NL = FP // 16

    def fill_z(t, _):
        rows[0][t // NL, pl.ds((t % NL) * 16, 16)] = zero16
        return 0
    lax.fori_loop(0, K * NL, fill_z, 0)

    def zchunk(t):
        sz = WCH if t < 15 else RPT - 15 * WCH
        return rows[0].at[pl.ds(0, sz)], nsum_sh.at[pl.ds(base + t * WCH, sz)]
    for t in range(16):
        zsrc, zdst = zchunk(t)
        pltpu.async_copy(zsrc, zdst, semg[t % 4])
    for t in range(16):
        zsrc, zdst = zchunk(t)
        pltpu.make_async_copy(zsrc, zdst, semg[t % 4]).wait()

    plsc.subcore_barrier()

    def idx_start(i, p):
        off = ebase + i * K
        pltpu.async_copy(ei_hbm.at[1, pl.ds(off, K)], colbs[p], semi[p])
        pltpu.async_copy(ei_hbm.at[0, pl.ds(off, K)], rowbs[p], semi[p])

    def idx_wait(i, p):
        off = ebase + i * K
        pltpu.make_async_copy(ei_hbm.at[1, pl.ds(off, K)], colbs[p], semi[p]).wait()
        pltpu.make_async_copy(ei_hbm.at[0, pl.ds(off, K)], rowbs[p], semi[p]).wait()

    def g_start(p):
        pltpu.async_copy(xp_hbm.at[colbs[p]], rows[p], semg[p])

    def g_wait(p):
        pltpu.make_async_copy(xp_hbm.at[colbs[p]], rows[p], semg[p]).wait()

    def s_start(p):
        pltpu.async_copy(rows[p], nsum_sh.at[rowbs[p]], sems[p], add=True)

    def s_wait(p):
        pltpu.make_async_copy(rows[p], nsum_sh.at[rowbs[p]], sems[p]).wait()

    idx_start(0, 0)
    idx_start(1, 1)
    idx_wait(0, 0)
    g_start(0)
    idx_start(2, 2)
    idx_wait(1, 1)
    g_start(1)
    g_wait(0)
    s_start(0)
    idx_start(3, 3)
    idx_wait(2, 2)
    g_start(2)
    g_wait(1)
    s_start(1)

    def body(u, _):
        t0 = 2 + 4 * u
        for r in range(4):
            t = t0 + r
            p1, p2, p3 = r, (r + 3) % 4, (r + 2) % 4
            s_wait(p1)
            idx_start(t + 2, p1)
            idx_wait(t + 1, p2)
            g_start(p2)
            g_wait(p3)
            s_start(p3)
        return 0
    lax.fori_loop(0, (NB - 6) // 4, body, 0)

    for t in (NB - 4, NB - 3):
        p1, p2, p3 = (t + 2) % 4, (t + 1) % 4, t % 4
        s_wait(p1)
        idx_start(t + 2, p1)
        idx_wait(t + 1, p2)
        g_start(p2)
        g_wait(p3)
        s_start(p3)
    t = NB - 2
    idx_wait(t + 1, (t + 1) % 4)
    g_start((t + 1) % 4)
    g_wait(t % 4)
    s_start(t % 4)
    t = NB - 1
    g_wait(t % 4)
    s_start(t % 4)
    for p in range(4):
        s_wait(p)

    plsc.subcore_barrier()

    def wchunk(t):
        sz = WCH if t < 15 else RPT - 15 * WCH
        return (nsum_sh.at[pl.ds(base + t * WCH, sz)],
                nsum_hbm.at[c, pl.ds(base + t * WCH, sz)])
    for t in range(16):
        ssrc, hdst = wchunk(t)
        pltpu.async_copy(ssrc, hdst, semg[t % 4])
    for t in range(16):
        ssrc, hdst = wchunk(t)
        pltpu.make_async_copy(ssrc, hdst, semg[t % 4]).wait()


BLK = 1000


def _tc_body(mult_ref, x_ref, ns_ref, w_ref, g1_ref, g2_ref,
             b1_ref, b2_ref, r_ref, hk_ref, out_ref):
    x = x_ref[...]
    nsum = ns_ref[0] + ns_ref[1]
    cnt = nsum[:, F:F + 1]
    neighbor = nsum[:, 0:F] / jnp.maximum(cnt, 1.0)

    def dot_t(a, w):
        return lax.dot_general(a, w, (((1,), (1,)), ((), ())),
                               preferred_element_type=jnp.float32)

    g = dot_t(x, g1_ref[...]) + dot_t(neighbor, g2_ref[...])
    gamma = jnp.where(g >= 0, g, 0.2 * g) + 1.0
    b = dot_t(x, b1_ref[...]) + dot_t(neighbor, b2_ref[...])
    beta = jnp.where(b >= 0, b, 0.2 * b)
    out = x + gamma * r_ref[...] + beta - neighbor
    out_ref[...] = out

    nb2 = neighbor + mult_ref[0, 0] * (out / (cnt + 1.0))
    hk_ref[:, 0:F] = dot_t(x, w_ref[...])
    hk_ref[:, F:2 * F] = dot_t(nb2, w_ref[...])


@jax.jit
def _tc_dense(mult, x, nsum_p, W, G1, G2, B1, B2, r):
    full = lambda shape: pl.BlockSpec(shape, lambda i: (0,) * len(shape))
    return pl.pallas_call(
        _tc_body,
        grid=(N // BLK,),
        in_specs=[
            pl.BlockSpec(memory_space=pltpu.SMEM),
            pl.BlockSpec((BLK, F), lambda i: (i, 0)),
            pl.BlockSpec((NC, BLK, FP), lambda i: (0, i, 0)),
            full((F, F)), full((F, F)), full((F, F)),
            full((F, F)), full((F, F)), full((1, F)),
        ],
        out_specs=[
            pl.BlockSpec((BLK, 2 * F), lambda i: (i, 0)),
            pl.BlockSpec((BLK, F), lambda i: (i, 0)),
        ],
        out_shape=[
            jax.ShapeDtypeStruct((N, 2 * F), jnp.float32),
            jax.ShapeDtypeStruct((N, F), jnp.float32),
        ],
    )(mult, x, nsum_p, W, G1, G2, B1, B2, r)


def kernel(x, edge_index, head, W, G1, G2, B1, B2, r):
    xp = jnp.concatenate([x, jnp.ones((N, FP - F), x.dtype)], axis=1)
    nsum_p = _sc_scatter(edge_index, xp)
    mult = jnp.where(head == 0, 1.0, 0.0).astype(jnp.float32).reshape(1, 1)
    hk, out = _tc_dense(mult, x, nsum_p, W, G1, G2, B1, B2, r)
    return hk, out


# speedup vs baseline: 1.0189x; 1.0189x over previous
import functools

import jax
import jax.numpy as jnp
from jax import lax
from jax.experimental import pallas as pl
from jax.experimental.pallas import tpu as pltpu
from jax.experimental.pallas import tpu_sc as plsc

N = 10000
F = 128
FP = 144
E = 320000

NC = 2
NS = 16
NW = NC * NS
EPT = E // NW
K = 40
NB = EPT // K
NP = 10112
RPT = NP // NS
WCH = 40

_mesh = plsc.VectorSubcoreMesh(core_axis_name="c", subcore_axis_name="s")


@functools.partial(
    pl.kernel,
    out_type=jax.ShapeDtypeStruct((NC, NP, FP), jnp.float32),
    mesh=_mesh,
    compiler_params=pltpu.CompilerParams(use_tc_tiling_on_sc=False),
    scratch_types=(
        tuple(pltpu.VMEM((K,), jnp.int32) for _ in range(4)),
        tuple(pltpu.VMEM((K,), jnp.int32) for _ in range(4)),
        tuple(pltpu.VMEM((K, FP), jnp.float32) for _ in range(4)),
        pltpu.VMEM_SHARED((NP, FP), jnp.float32),
        tuple(pltpu.SemaphoreType.DMA for _ in range(4)),
        tuple(pltpu.SemaphoreType.DMA for _ in range(4)),
        tuple(pltpu.SemaphoreType.DMA for _ in range(4)),
    ),
)
def _sc_scatter(ei_hbm, xp_hbm, nsum_hbm,
                colbs, rowbs, rows, nsum_sh, semi, semg, sems):
    c = lax.axis_index("c")
    s = lax.axis_index("s")
    wid = s * NC + c
    ebase = wid * EPT
    base = s * RPT

    zero16 = jnp.zeros((16,), jnp.float32)
    NL = FP // 16

    def fill_z(t, _):
        rows[0][t // NL, pl.ds((t % NL) * 16, 16)] = zero16
        return 0
    lax.fori_loop(0, K * NL, fill_z, 0)

    def zchunk(t):
        sz = WCH if t < 15 else RPT - 15 * WCH
        return rows[0].at[pl.ds(0, sz)], nsum_sh.at[pl.ds(base + t * WCH, sz)]
    for t in range(16):
        zsrc, zdst = zchunk(t)
        pltpu.async_copy(zsrc, zdst, semg[t % 4])
    for t in range(16):
        zsrc, zdst = zchunk(t)
        pltpu.make_async_copy(zsrc, zdst, semg[t % 4]).wait()

    plsc.subcore_barrier()

    def idx_start(i, p):
        off = ebase + i * K
        pltpu.async_copy(ei_hbm.at[1, pl.ds(off, K)], colbs[p], semi[p])
        pltpu.async_copy(ei_hbm.at[0, pl.ds(off, K)], rowbs[p], semi[p])

    def idx_wait(i, p):
        off = ebase + i * K
        pltpu.make_async_copy(ei_hbm.at[1, pl.ds(off, K)], colbs[p], semi[p]).wait()
        pltpu.make_async_copy(ei_hbm.at[0, pl.ds(off, K)], rowbs[p], semi[p]).wait()

    def g_start(p):
        pltpu.async_copy(xp_hbm.at[colbs[p]], rows[p], semg[p])

    def g_wait(p):
        pltpu.make_async_copy(xp_hbm.at[colbs[p]], rows[p], semg[p]).wait()

    def s_start(p):
        pltpu.async_copy(rows[p], nsum_sh.at[rowbs[p]], sems[p], add=True)

    def s_wait(p):
        pltpu.make_async_copy(rows[p], nsum_sh.at[rowbs[p]], sems[p]).wait()

    idx_start(0, 0)
    idx_start(1, 1)
    idx_wait(0, 0)
    g_start(0)
    idx_start(2, 2)
    idx_wait(1, 1)
    g_start(1)
    g_wait(0)
    s_start(0)
    idx_start(3, 3)
    idx_wait(2, 2)
    g_start(2)
    g_wait(1)
    s_start(1)

    def body(u, _):
        t0 = 2 + 4 * u
        for r in range(4):
            t = t0 + r
            p1, p2, p3 = r, (r + 3) % 4, (r + 2) % 4
            s_wait(p1)
            idx_start(t + 2, p1)
            idx_wait(t + 1, p2)
            g_start(p2)
            g_wait(p3)
            s_start(p3)
        return 0
    lax.fori_loop(0, (NB - 6) // 4, body, 0)

    for t in (NB - 4, NB - 3):
        p1, p2, p3 = (t + 2) % 4, (t + 1) % 4, t % 4
        s_wait(p1)
        idx_start(t + 2, p1)
        idx_wait(t + 1, p2)
        g_start(p2)
        g_wait(p3)
        s_start(p3)
    t = NB - 2
    idx_wait(t + 1, (t + 1) % 4)
    g_start((t + 1) % 4)
    g_wait(t % 4)
    s_start(t % 4)
    t = NB - 1
    g_wait(t % 4)
    s_start(t % 4)
    for p in range(4):
        s_wait(p)

    plsc.subcore_barrier()

    def wchunk(t, p):
        sz = WCH if t < 15 else RPT - 15 * WCH
        return (rows[p].at[pl.ds(0, sz)],
                nsum_sh.at[pl.ds(base + t * WCH, sz)],
                nsum_hbm.at[c, pl.ds(base + t * WCH, sz)])
    for t in range(16):
        p = t % 4
        if t >= 4:
            vsrc, _, hdst = wchunk(t - 4, p)
            pltpu.make_async_copy(vsrc, hdst, semg[p]).wait()
        vsrc, ssrc, hdst = wchunk(t, p)
        pltpu.sync_copy(ssrc, vsrc)
        pltpu.async_copy(vsrc, hdst, semg[p])
    for t in range(12, 16):
        p = t % 4
        vsrc, _, hdst = wchunk(t, p)
        pltpu.make_async_copy(vsrc, hdst, semg[p]).wait()


BLK = 2000


def _tc_body(mult_ref, x_ref, ns_ref, w_ref, g1_ref, g2_ref,
             b1_ref, b2_ref, r_ref, hk_ref, out_ref):
    x = x_ref[...]
    nsum = ns_ref[0] + ns_ref[1]
    cnt = nsum[:, F:F + 1]
    neighbor = nsum[:, 0:F] / jnp.maximum(cnt, 1.0)

    def dot_t(a, w):
        return lax.dot_general(a, w, (((1,), (1,)), ((), ())),
                               preferred_element_type=jnp.float32)

    g = dot_t(x, g1_ref[...]) + dot_t(neighbor, g2_ref[...])
    gamma = jnp.where(g >= 0, g, 0.2 * g) + 1.0
    b = dot_t(x, b1_ref[...]) + dot_t(neighbor, b2_ref[...])
    beta = jnp.where(b >= 0, b, 0.2 * b)
    out = x + gamma * r_ref[...] + beta - neighbor
    out_ref[...] = out

    nb2 = neighbor + mult_ref[0, 0] * (out / (cnt + 1.0))
    hk_ref[:, 0:F] = dot_t(x, w_ref[...])
    hk_ref[:, F:2 * F] = dot_t(nb2, w_ref[...])


@jax.jit
def _tc_dense(mult, x, nsum_p, W, G1, G2, B1, B2, r):
    full = lambda shape: pl.BlockSpec(shape, lambda i: (0,) * len(shape))
    return pl.pallas_call(
        _tc_body,
        grid=(N // BLK,),
        in_specs=[
            pl.BlockSpec(memory_space=pltpu.SMEM),
            pl.BlockSpec((BLK, F), lambda i: (i, 0)),
            pl.BlockSpec((NC, BLK, FP), lambda i: (0, i, 0)),
            full((F, F)), full((F, F)), full((F, F)),
            full((F, F)), full((F, F)), full((1, F)),
        ],
        out_specs=[
            pl.BlockSpec((BLK, 2 * F), lambda i: (i, 0)),
            pl.BlockSpec((BLK, F), lambda i: (i, 0)),
        ],
        out_shape=[
            jax.ShapeDtypeStruct((N, 2 * F), jnp.float32),
            jax.ShapeDtypeStruct((N, F), jnp.float32),
        ],
    )(mult, x, nsum_p, W, G1, G2, B1, B2, r)


def kernel(x, edge_index, head, W, G1, G2, B1, B2, r):
    xp = jnp.concatenate([x, jnp.ones((N, FP - F), x.dtype)], axis=1)
    nsum_p = _sc_scatter(edge_index, xp)
    mult = jnp.where(head == 0, 1.0, 0.0).astype(jnp.float32).reshape(1, 1)
    hk, out = _tc_dense(mult, x, nsum_p, W, G1, G2, B1, B2, r)
    return hk, out
